# baseline (speedup 1.0000x reference)
import functools

import jax
import jax.numpy as jnp
from jax import lax
from jax.experimental import pallas as pl
from jax.experimental.pallas import tpu as pltpu
from jax.experimental.pallas import tpu_sc as plsc

B = 4096
H = 1024
EMB = H // 2
NOPS = 8
BT = 256
PB = B + (NOPS - 1) * BT
W = PB // BT
RS, LS = 32, 128


def _sc_row_gather(table, idx):
    n_rows, d = table.shape
    info = plsc.get_sparse_core_info()
    nw = info.num_cores * info.num_subcores
    b_per_w = idx.shape[0] // nw
    ch = 64
    mesh = plsc.VectorSubcoreMesh(core_axis_name="c", subcore_axis_name="s")

    @functools.partial(
        pl.kernel,
        mesh=mesh,
        out_type=jax.ShapeDtypeStruct((idx.shape[0], d), jnp.float32),
        scratch_types=[
            pltpu.VMEM((ch,), jnp.int32),
            pltpu.VMEM((ch, d), jnp.float32),
            pltpu.SemaphoreType.DMA,
        ],
    )
    def gather_k(table_hbm, idx_hbm, out_hbm, idx_v, rows_v, sem):
        wid = lax.axis_index("s") * info.num_cores + lax.axis_index("c")
        base = wid * b_per_w
        for c in range(b_per_w // ch):
            off = base + c * ch
            pltpu.sync_copy(idx_hbm.at[pl.ds(off, ch)], idx_v)
            pltpu.async_copy(table_hbm.at[idx_v], rows_v, sem).wait()
            pltpu.sync_copy(rows_v, out_hbm.at[pl.ds(off, ch)])

    return gather_k(table, idx)


def _sc_row_scatter(rows, idx, out_rows):
    n_rows, d = rows.shape
    info = plsc.get_sparse_core_info()
    nw = info.num_cores * info.num_subcores
    b_per_w = n_rows // nw
    ch = 64
    mesh = plsc.VectorSubcoreMesh(core_axis_name="c", subcore_axis_name="s")

    @functools.partial(
        pl.kernel,
        mesh=mesh,
        out_type=jax.ShapeDtypeStruct((out_rows, d), jnp.float32),
        scratch_types=[
            pltpu.VMEM((ch,), jnp.int32),
            pltpu.VMEM((ch, d), jnp.float32),
            pltpu.SemaphoreType.DMA,
        ],
    )
    def scatter_k(rows_hbm, idx_hbm, out_hbm, idx_v, rows_v, sem):
        wid = lax.axis_index("s") * info.num_cores + lax.axis_index("c")
        base = wid * b_per_w
        for c in range(b_per_w // ch):
            off = base + c * ch
            pltpu.sync_copy(idx_hbm.at[pl.ds(off, ch)], idx_v)
            pltpu.sync_copy(rows_hbm.at[pl.ds(off, ch)], rows_v)
            pltpu.async_copy(rows_v, out_hbm.at[idx_v], sem).wait()

    return scatter_k(rows, idx)


def _shift_lanes(x, k):
    return jnp.concatenate([jnp.zeros((x.shape[0], k), x.dtype), x[:, :-k]], axis=1)


def _shift_subl(x, k):
    return jnp.concatenate([jnp.zeros((k, x.shape[1]), x.dtype), x[:-k, :]], axis=0)


def _lane_incl_scan(x):
    for k in (1, 2, 4, 8, 16, 32, 64):
        x = x + _shift_lanes(x, k)
    return x


def _meta_body(ops_ref, pos_ref, meta_ref):
    ops = ops_ref[...]
    lane_iota = lax.broadcasted_iota(jnp.int32, (1, LS), 1)
    within = jnp.zeros((RS, LS), jnp.int32)
    sizes_v = jnp.zeros((1, LS), jnp.int32)
    for o in range(NOPS):
        m = (ops == o).astype(jnp.int32)
        c = _lane_incl_scan(m)
        rt = c[:, LS - 1:LS]
        s = rt
        for k in (1, 2, 4, 8, 16):
            s = s + _shift_subl(s, k)
        within = within + (c - 1 + (s - rt)) * m
        sizes_v = sizes_v + s[RS - 1:RS, 0:1] * (lane_iota == o).astype(jnp.int32)
    ct = (sizes_v + (BT - 1)) // BT
    cum_ct = _lane_incl_scan(ct)
    pstart_v = (cum_ct - ct) * BT
    pos = within
    for o in range(NOPS):
        pos = pos + (ops == o).astype(jnp.int32) * pstart_v[0:1, o:o + 1]
    pos_ref[...] = pos
    total = cum_ct[0:1, NOPS - 1:NOPS]
    w_eff = jnp.minimum(lane_iota, total - 1)
    ug = jnp.zeros((1, LS), jnp.int32)
    for o in range(NOPS - 1):
        ug = ug + (cum_ct[0:1, o:o + 1] <= w_eff).astype(jnp.int32)
    meta_ref[...] = jnp.concatenate([ug, jnp.zeros((NOPS - 1, LS), jnp.int32)], axis=0)


def _make_metadata(sampled_ops):
    pos2d, meta = pl.pallas_call(
        _meta_body,
        in_specs=[pl.BlockSpec((RS, LS), lambda: (0, 0))],
        out_specs=[pl.BlockSpec((RS, LS), lambda: (0, 0)),
                   pl.BlockSpec((NOPS, LS), lambda: (0, 0))],
        out_shape=[jax.ShapeDtypeStruct((RS, LS), jnp.int32),
                   jax.ShapeDtypeStruct((NOPS, LS), jnp.int32)],
    )(sampled_ops.reshape(RS, LS))
    return pos2d.reshape(B), meta


def _bias_body(emb_ref, W1b_ref, b1_ref, out_ref):
    out_ref[0] = b1_ref[0] + jnp.dot(emb_ref[0], W1b_ref[0],
                                     preferred_element_type=jnp.float32)


def _bias_eff(op_emb, W1, b1):
    emb3 = op_emb.reshape(NOPS, 1, EMB)
    b13 = b1.reshape(NOPS, 1, H)
    return pl.pallas_call(
        _bias_body,
        grid=(NOPS,),
        in_specs=[
            pl.BlockSpec((1, 1, EMB), lambda g: (g, 0, 0)),
            pl.BlockSpec((1, EMB, H), lambda g: (g, 2, 0)),
            pl.BlockSpec((1, 1, H), lambda g: (g, 0, 0)),
        ],
        out_specs=pl.BlockSpec((1, 1, H), lambda g: (g, 0, 0)),
        out_shape=jax.ShapeDtypeStruct((NOPS, 1, H), jnp.float32),
        compiler_params=pltpu.CompilerParams(
            dimension_semantics=("arbitrary",),
        ),
    )(emb3, W1, b13)


def _mlp_body(meta_ref, x_ref, W1_ref, b1_ref, W2_ref, b2_ref, out_ref):
    h = jnp.maximum(
        jnp.dot(x_ref[...], W1_ref[0], preferred_element_type=jnp.float32) + b1_ref[0], 0.0)
    out_ref[...] = jnp.maximum(
        jnp.dot(h, W2_ref[0], preferred_element_type=jnp.float32) + b2_ref[0], 0.0)


def _grouped_mlp(x_sorted, op_emb, W1, b1, W2, b2, meta):
    b1_eff = _bias_eff(op_emb, W1, b1)
    b23 = b2.reshape(NOPS, 1, H)
    grid_spec = pltpu.PrefetchScalarGridSpec(
        num_scalar_prefetch=1,
        grid=(W,),
        in_specs=[
            pl.BlockSpec((BT, H), lambda w, M: (w, 0)),
            pl.BlockSpec((1, H, H), lambda w, M: (M[0, w], 0, 0)),
            pl.BlockSpec((1, 1, H), lambda w, M: (M[0, w], 0, 0)),
            pl.BlockSpec((1, H, H), lambda w, M: (M[0, w], 0, 0)),
            pl.BlockSpec((1, 1, H), lambda w, M: (M[0, w], 0, 0)),
        ],
        out_specs=pl.BlockSpec((BT, H), lambda w, M: (w, 0)),
    )
    return pl.pallas_call(
        _mlp_body,
        grid_spec=grid_spec,
        out_shape=jax.ShapeDtypeStruct((PB, H), jnp.float32),
        compiler_params=pltpu.CompilerParams(
            dimension_semantics=("arbitrary",),
        ),
    )(meta, x_sorted, W1, b1_eff, W2, b23)


def kernel(x, op_emb, W1, b1, W2, b2, sampled_ops):
    sampled_ops = sampled_ops.astype(jnp.int32)
    pos, meta = _make_metadata(sampled_ops)
    x_sorted = _sc_row_scatter(x, pos, PB)
    y_sorted = _grouped_mlp(x_sorted, op_emb, W1, b1, W2, b2, meta)
    return _sc_row_gather(y_sorted, pos)


# speedup vs baseline: 1.0113x; 1.0113x over previous
import functools

import jax
import jax.numpy as jnp
from jax import lax
from jax.experimental import pallas as pl
from jax.experimental.pallas import tpu as pltpu
from jax.experimental.pallas import tpu_sc as plsc

B = 4096
H = 1024
EMB = H // 2
NOPS = 8
BT = 256
T = B // BT
W = T + NOPS - 1
RS, LS = 32, 128


def _sc_row_gather(table, idx):
    n_rows, d = table.shape
    info = plsc.get_sparse_core_info()
    nw = info.num_cores * info.num_subcores
    b_per_w = idx.shape[0] // nw
    ch = 64
    mesh = plsc.VectorSubcoreMesh(core_axis_name="c", subcore_axis_name="s")

    @functools.partial(
        pl.kernel,
        mesh=mesh,
        out_type=jax.ShapeDtypeStruct((idx.shape[0], d), jnp.float32),
        scratch_types=[
            pltpu.VMEM((ch,), jnp.int32),
            pltpu.VMEM((ch, d), jnp.float32),
            pltpu.SemaphoreType.DMA,
        ],
    )
    def gather_k(table_hbm, idx_hbm, out_hbm, idx_v, rows_v, sem):
        wid = lax.axis_index("s") * info.num_cores + lax.axis_index("c")
        base = wid * b_per_w
        for c in range(b_per_w // ch):
            off = base + c * ch
            pltpu.sync_copy(idx_hbm.at[pl.ds(off, ch)], idx_v)
            pltpu.async_copy(table_hbm.at[idx_v], rows_v, sem).wait()
            pltpu.sync_copy(rows_v, out_hbm.at[pl.ds(off, ch)])

    return gather_k(table, idx)


def _sc_row_scatter(rows, idx):
    n_rows, d = rows.shape
    info = plsc.get_sparse_core_info()
    nw = info.num_cores * info.num_subcores
    b_per_w = n_rows // nw
    ch = 64
    mesh = plsc.VectorSubcoreMesh(core_axis_name="c", subcore_axis_name="s")

    @functools.partial(
        pl.kernel,
        mesh=mesh,
        out_type=jax.ShapeDtypeStruct((n_rows, d), jnp.float32),
        scratch_types=[
            pltpu.VMEM((ch,), jnp.int32),
            pltpu.VMEM((ch, d), jnp.float32),
            pltpu.SemaphoreType.DMA,
        ],
    )
    def scatter_k(rows_hbm, idx_hbm, out_hbm, idx_v, rows_v, sem):
        wid = lax.axis_index("s") * info.num_cores + lax.axis_index("c")
        base = wid * b_per_w
        for c in range(b_per_w // ch):
            off = base + c * ch
            pltpu.sync_copy(idx_hbm.at[pl.ds(off, ch)], idx_v)
            pltpu.sync_copy(rows_hbm.at[pl.ds(off, ch)], rows_v)
            pltpu.async_copy(rows_v, out_hbm.at[idx_v], sem).wait()

    return scatter_k(rows, idx)


def _shift_lanes(x, k):
    return jnp.concatenate([jnp.zeros((x.shape[0], k), x.dtype), x[:, :-k]], axis=1)


def _shift_subl(x, k):
    return jnp.concatenate([jnp.zeros((k, x.shape[1]), x.dtype), x[:-k, :]], axis=0)


def _lane_incl_scan(x):
    for k in (1, 2, 4, 8, 16, 32, 64):
        x = x + _shift_lanes(x, k)
    return x


def _meta_body(ops_ref, pos_ref, meta_ref):
    ops = ops_ref[...]
    lane_iota = lax.broadcasted_iota(jnp.int32, (1, LS), 1)
    within = jnp.zeros((RS, LS), jnp.int32)
    sizes_v = jnp.zeros((1, LS), jnp.int32)
    for o in range(NOPS):
        m = (ops == o).astype(jnp.int32)
        c = _lane_incl_scan(m)
        rt = c[:, LS - 1:LS]
        s = rt
        for k in (1, 2, 4, 8, 16):
            s = s + _shift_subl(s, k)
        within = within + (c - 1 + (s - rt)) * m
        sizes_v = sizes_v + s[RS - 1:RS, 0:1] * (lane_iota == o).astype(jnp.int32)
    starts_v = _lane_incl_scan(sizes_v) - sizes_v
    ends_v = starts_v + sizes_v
    pos = within
    for o in range(NOPS):
        pos = pos + (ops == o).astype(jnp.int32) * starts_v[0:1, o:o + 1]
    pos_ref[...] = pos

    first_tile = starts_v // BT
    last_tile = jnp.maximum(ends_v - 1, 0) // BT
    ntiles = jnp.where(sizes_v > 0, last_tile - first_tile + 1, 0)
    cum_tiles = _lane_incl_scan(ntiles)
    total = cum_tiles[0:1, NOPS - 1:NOPS]
    w_eff = jnp.minimum(lane_iota, total - 1)
    ug = jnp.zeros((1, LS), jnp.int32)
    for o in range(NOPS - 1):
        ug = ug + (cum_tiles[0:1, o:o + 1] <= w_eff).astype(jnp.int32)
    ut = jnp.zeros((1, LS), jnp.int32)
    for o in range(NOPS):
        val = first_tile[0:1, o:o + 1] + w_eff - (cum_tiles[0:1, o:o + 1] - ntiles[0:1, o:o + 1])
        ut = ut + (ug == o).astype(jnp.int32) * val
    zero = jnp.zeros((NOPS - 4, LS), jnp.int32)
    meta_ref[...] = jnp.concatenate([starts_v, ends_v, ug, ut, zero], axis=0)


def _make_metadata(sampled_ops):
    pos2d, meta = pl.pallas_call(
        _meta_body,
        in_specs=[pl.BlockSpec((RS, LS), lambda: (0, 0))],
        out_specs=[pl.BlockSpec((RS, LS), lambda: (0, 0)),
                   pl.BlockSpec((NOPS, LS), lambda: (0, 0))],
        out_shape=[jax.ShapeDtypeStruct((RS, LS), jnp.int32),
                   jax.ShapeDtypeStruct((NOPS, LS), jnp.int32)],
    )(sampled_ops.reshape(RS, LS))
    return pos2d.reshape(B), meta


def _mlp_body(meta_ref, x_ref, W1_ref, W1b_ref, emb_ref, b1_ref, W2_ref,
              b2_ref, out_ref, bias_scr):
    w = pl.program_id(0)
    g = meta_ref[2, w]
    t = meta_ref[3, w]

    @pl.when(jnp.logical_or(w == 0, meta_ref[2, jnp.maximum(w - 1, 0)] != g))
    def _():
        bias_scr[...] = b1_ref[0] + jnp.dot(emb_ref[0], W1b_ref[0],
                                            preferred_element_type=jnp.float32)

    h = jnp.maximum(
        jnp.dot(x_ref[...], W1_ref[0], preferred_element_type=jnp.float32)
        + bias_scr[...], 0.0)
    y = jnp.maximum(
        jnp.dot(h, W2_ref[0], preferred_element_type=jnp.float32) + b2_ref[0], 0.0)
    rows = t * BT + lax.broadcasted_iota(jnp.int32, (BT, 1), 0)
    mask = (rows >= meta_ref[0, g]) & (rows < meta_ref[1, g])
    out_ref[...] = jnp.where(mask, y, out_ref[...])


def _grouped_mlp(x_sorted, op_emb, W1, b1, W2, b2, meta):
    emb3 = op_emb.reshape(NOPS, 1, EMB)
    b13 = b1.reshape(NOPS, 1, H)
    b23 = b2.reshape(NOPS, 1, H)
    grid_spec = pltpu.PrefetchScalarGridSpec(
        num_scalar_prefetch=1,
        grid=(W,),
        in_specs=[
            pl.BlockSpec((BT, H), lambda w, M: (M[3, w], 0)),
            pl.BlockSpec((1, H, H), lambda w, M: (M[2, w], 0, 0)),
            pl.BlockSpec((1, EMB, H), lambda w, M: (M[2, w], 2, 0)),
            pl.BlockSpec((1, 1, EMB), lambda w, M: (M[2, w], 0, 0)),
            pl.BlockSpec((1, 1, H), lambda w, M: (M[2, w], 0, 0)),
            pl.BlockSpec((1, H, H), lambda w, M: (M[2, w], 0, 0)),
            pl.BlockSpec((1, 1, H), lambda w, M: (M[2, w], 0, 0)),
        ],
        out_specs=pl.BlockSpec((BT, H), lambda w, M: (M[3, w], 0)),
        scratch_shapes=[pltpu.VMEM((1, H), jnp.float32)],
    )
    return pl.pallas_call(
        _mlp_body,
        grid_spec=grid_spec,
        out_shape=jax.ShapeDtypeStruct((B, H), jnp.float32),
        compiler_params=pltpu.CompilerParams(
            dimension_semantics=("arbitrary",),
        ),
    )(meta, x_sorted, W1, W1, emb3, b13, W2, b23)


def kernel(x, op_emb, W1, b1, W2, b2, sampled_ops):
    sampled_ops = sampled_ops.astype(jnp.int32)
    pos, meta = _make_metadata(sampled_ops)
    x_sorted = _sc_row_scatter(x, pos)
    y_sorted = _grouped_mlp(x_sorted, op_emb, W1, b1, W2, b2, meta)
    return _sc_row_gather(y_sorted, pos)


# speedup vs baseline: 1.0291x; 1.0177x over previous
import functools

import jax
import jax.numpy as jnp
from jax import lax
from jax.experimental import pallas as pl
from jax.experimental.pallas import tpu as pltpu
from jax.experimental.pallas import tpu_sc as plsc

B = 4096
H = 1024
EMB = H // 2
NOPS = 8
BT = 256
T = B // BT
W = T + NOPS - 1
RS, LS = 32, 128


def _sc_row_gather(table, idx):
    n_rows, d = table.shape
    info = plsc.get_sparse_core_info()
    nw = info.num_cores * info.num_subcores
    b_per_w = idx.shape[0] // nw
    ch = 64
    mesh = plsc.VectorSubcoreMesh(core_axis_name="c", subcore_axis_name="s")

    @functools.partial(
        pl.kernel,
        mesh=mesh,
        out_type=jax.ShapeDtypeStruct((idx.shape[0], d), jnp.float32),
        scratch_types=[
            pltpu.VMEM((ch,), jnp.int32),
            pltpu.VMEM((ch, d), jnp.float32),
            pltpu.SemaphoreType.DMA,
        ],
    )
    def gather_k(table_hbm, idx_hbm, out_hbm, idx_v, rows_v, sem):
        wid = lax.axis_index("s") * info.num_cores + lax.axis_index("c")
        base = wid * b_per_w
        for c in range(b_per_w // ch):
            off = base + c * ch
            pltpu.sync_copy(idx_hbm.at[pl.ds(off, ch)], idx_v)
            pltpu.async_copy(table_hbm.at[idx_v], rows_v, sem).wait()
            pltpu.sync_copy(rows_v, out_hbm.at[pl.ds(off, ch)])

    return gather_k(table, idx)


def _sc_row_scatter(rows, idx):
    n_rows, d = rows.shape
    info = plsc.get_sparse_core_info()
    nw = info.num_cores * info.num_subcores
    b_per_w = n_rows // nw
    ch = 64
    mesh = plsc.VectorSubcoreMesh(core_axis_name="c", subcore_axis_name="s")

    @functools.partial(
        pl.kernel,
        mesh=mesh,
        out_type=jax.ShapeDtypeStruct((n_rows, d), jnp.float32),
        scratch_types=[
            pltpu.VMEM((ch,), jnp.int32),
            pltpu.VMEM((ch, d), jnp.float32),
            pltpu.SemaphoreType.DMA,
        ],
    )
    def scatter_k(rows_hbm, idx_hbm, out_hbm, idx_v, rows_v, sem):
        wid = lax.axis_index("s") * info.num_cores + lax.axis_index("c")
        base = wid * b_per_w
        for c in range(b_per_w // ch):
            off = base + c * ch
            pltpu.sync_copy(idx_hbm.at[pl.ds(off, ch)], idx_v)
            pltpu.sync_copy(rows_hbm.at[pl.ds(off, ch)], rows_v)
            pltpu.async_copy(rows_v, out_hbm.at[idx_v], sem).wait()

    return scatter_k(rows, idx)


def _shift_lanes(x, k):
    return jnp.concatenate([jnp.zeros((x.shape[0], k), x.dtype), x[:, :-k]], axis=1)


def _shift_subl(x, k):
    return jnp.concatenate([jnp.zeros((k, x.shape[1]), x.dtype), x[:-k, :]], axis=0)


def _lane_incl_scan(x):
    for k in (1, 2, 4, 8, 16, 32, 64):
        x = x + _shift_lanes(x, k)
    return x


def _meta_body(ops_ref, pos_ref, meta_ref):
    ops = ops_ref[...]
    lane_iota = lax.broadcasted_iota(jnp.int32, (1, LS), 1)
    within = jnp.zeros((RS, LS), jnp.int32)
    sizes_v = jnp.zeros((1, LS), jnp.int32)
    for o in range(NOPS):
        m = (ops == o).astype(jnp.int32)
        c = _lane_incl_scan(m)
        rt = c[:, LS - 1:LS]
        s = rt
        for k in (1, 2, 4, 8, 16):
            s = s + _shift_subl(s, k)
        within = within + (c - 1 + (s - rt)) * m
        sizes_v = sizes_v + s[RS - 1:RS, 0:1] * (lane_iota == o).astype(jnp.int32)
    starts_v = _lane_incl_scan(sizes_v) - sizes_v
    ends_v = starts_v + sizes_v
    pos = within
    for o in range(NOPS):
        pos = pos + (ops == o).astype(jnp.int32) * starts_v[0:1, o:o + 1]
    pos_ref[...] = pos

    first_tile = starts_v // BT
    last_tile = jnp.maximum(ends_v - 1, 0) // BT
    ntiles = jnp.where(sizes_v > 0, last_tile - first_tile + 1, 0)
    cum_tiles = _lane_incl_scan(ntiles)
    total = cum_tiles[0:1, NOPS - 1:NOPS]
    w_eff = jnp.minimum(lane_iota, total - 1)
    ug = jnp.zeros((1, LS), jnp.int32)
    for o in range(NOPS - 1):
        ug = ug + (cum_tiles[0:1, o:o + 1] <= w_eff).astype(jnp.int32)
    ut = jnp.zeros((1, LS), jnp.int32)
    for o in range(NOPS):
        val = first_tile[0:1, o:o + 1] + w_eff - (cum_tiles[0:1, o:o + 1] - ntiles[0:1, o:o + 1])
        ut = ut + (ug == o).astype(jnp.int32) * val
    zero = jnp.zeros((NOPS - 4, LS), jnp.int32)
    meta_ref[...] = jnp.concatenate([starts_v, ends_v, ug, ut, zero], axis=0)


def _make_metadata(sampled_ops):
    pos2d, meta = pl.pallas_call(
        _meta_body,
        in_specs=[pl.BlockSpec((RS, LS), lambda: (0, 0))],
        out_specs=[pl.BlockSpec((RS, LS), lambda: (0, 0)),
                   pl.BlockSpec((NOPS, LS), lambda: (0, 0))],
        out_shape=[jax.ShapeDtypeStruct((RS, LS), jnp.int32),
                   jax.ShapeDtypeStruct((NOPS, LS), jnp.int32)],
    )(sampled_ops.reshape(RS, LS))
    return pos2d.reshape(B), meta


def _bias_body(emb_ref, W1b_ref, b1_ref, out_ref):
    out_ref[0] = b1_ref[0] + jnp.dot(emb_ref[0], W1b_ref[0],
                                     preferred_element_type=jnp.float32)


def _bias_eff(op_emb, W1, b1):
    emb3 = op_emb.reshape(NOPS, 1, EMB)
    b13 = b1.reshape(NOPS, 1, H)
    return pl.pallas_call(
        _bias_body,
        grid=(NOPS,),
        in_specs=[
            pl.BlockSpec((1, 1, EMB), lambda g: (g, 0, 0)),
            pl.BlockSpec((1, EMB, H), lambda g: (g, 2, 0)),
            pl.BlockSpec((1, 1, H), lambda g: (g, 0, 0)),
        ],
        out_specs=pl.BlockSpec((1, 1, H), lambda g: (g, 0, 0)),
        out_shape=jax.ShapeDtypeStruct((NOPS, 1, H), jnp.float32),
        compiler_params=pltpu.CompilerParams(
            dimension_semantics=("arbitrary",),
        ),
    )(emb3, W1, b13)


def _mlp_body(meta_ref, x_ref, W1_ref, b1_ref, W2_ref, b2_ref, out_ref):
    w = pl.program_id(0)
    g = meta_ref[2, w]
    t = meta_ref[3, w]
    xb = x_ref[...]
    h = jnp.maximum(
        jnp.dot(xb, W1_ref[0], preferred_element_type=jnp.float32) + b1_ref[0], 0.0)
    y = jnp.maximum(
        jnp.dot(h, W2_ref[0], preferred_element_type=jnp.float32) + b2_ref[0], 0.0)
    rows = t * BT + lax.broadcasted_iota(jnp.int32, (BT, 1), 0)
    mask = (rows >= meta_ref[0, g]) & (rows < meta_ref[1, g])
    out_ref[...] = jnp.where(mask, y, out_ref[...])


def _grouped_mlp(x_sorted, op_emb, W1, b1, W2, b2, meta):
    b1_eff = _bias_eff(op_emb, W1, b1)
    b23 = b2.reshape(NOPS, 1, H)
    grid_spec = pltpu.PrefetchScalarGridSpec(
        num_scalar_prefetch=1,
        grid=(W,),
        in_specs=[
            pl.BlockSpec((BT, H), lambda w, M: (M[3, w], 0)),
            pl.BlockSpec((1, H, H), lambda w, M: (M[2, w], 0, 0)),
            pl.BlockSpec((1, 1, H), lambda w, M: (M[2, w], 0, 0)),
            pl.BlockSpec((1, H, H), lambda w, M: (M[2, w], 0, 0)),
            pl.BlockSpec((1, 1, H), lambda w, M: (M[2, w], 0, 0)),
        ],
        out_specs=pl.BlockSpec((BT, H), lambda w, M: (M[3, w], 0)),
    )
    return pl.pallas_call(
        _mlp_body,
        grid_spec=grid_spec,
        out_shape=jax.ShapeDtypeStruct((B, H), jnp.float32),
        compiler_params=pltpu.CompilerParams(
            dimension_semantics=("arbitrary",),
        ),
    )(meta, x_sorted, W1, b1_eff, W2, b23)


def kernel(x, op_emb, W1, b1, W2, b2, sampled_ops):
    sampled_ops = sampled_ops.astype(jnp.int32)
    pos, meta = _make_metadata(sampled_ops)
    x_sorted = _sc_row_scatter(x, pos)
    y_sorted = _grouped_mlp(x_sorted, op_emb, W1, b1, W2, b2, meta)
    return _sc_row_gather(y_sorted, pos)
